# baseline (speedup 1.0000x reference)
import jax
import jax.numpy as jnp
from jax import lax
from jax.experimental import pallas as pl
from jax.experimental.pallas import tpu as pltpu
from jax.experimental.pallas import tpu_sc as plsc

SCORE_TH = 0.05
NMS_TH = 0.5
MAXDET = 300
N = 20000
C = 80
NB = 2
NS = 16
L = 16
GRAN = N // L
CHUNK_G = 4
RCHUNK = CHUNK_G * L
NCHUNK = 20
NEG = -3.0e38
TOPP = 304
QN = 5000
NHALF = 76


def _vecmax5(vecs):
    m01 = jnp.maximum(vecs[0], vecs[1])
    m23 = jnp.maximum(vecs[2], vecs[3])
    return jnp.maximum(jnp.maximum(m01, m23), vecs[4])


def _fd_body(boxt_hbm, cls_hbm, ob_hbm, os_hbm, ol_hbm,
             cls_a, cls_b, lb64, scores_loc, cm_loc, loc_sc, loc_idx,
             msc, midx, labels_f,
             top_idx, top_sc, labels_v, fb_soa, box_buf,
             iou_loc, iou_half, alive, stage_b, stage_s, stage_l, nv_smem,
             locsc_sh, locidx_sh, labels_sh, fb_sh, iou_sh,
             sem_a, sem_b, sem_c):
    c = lax.axis_index("c")
    s = lax.axis_index("s")
    lane = lax.iota(jnp.int32, L)
    lane0 = lane == 0

    def _sst(ref, idx, val):
        plsc.store_scatter(ref, [jnp.full((L,), idx, jnp.int32)],
                           jnp.full((L,), val), mask=lane0)

    def _sstv(ref, idx, vec):
        plsc.store_scatter(ref, [jnp.full((L,), idx, jnp.int32)], vec,
                           mask=lane0)

    def _sldv(ref, idx):
        return plsc.load_gather(ref, [jnp.full((L,), idx, jnp.int32)])

    def _scal(x):
        return x[0] if getattr(x, "ndim", 0) else x

    g_lo = (GRAN * s) // NS
    g_hi = (GRAN * (s + 1)) // NS

    def _base(k):
        return jnp.minimum(g_lo + k * CHUNK_G, g_hi - CHUNK_G)

    def _issue(k, buf, sem):
        row0 = c * N + _base(k) * L
        return pltpu.async_copy(cls_hbm.at[pl.ds(row0, RCHUNK)], buf, sem)

    def _process(k, buf):
        loff = (_base(k) - g_lo) * L

        def row_body(r, _):
            vecs = [buf[r, pl.ds(L * j, L)] for j in range(5)]
            best = vecs[0]
            bc = lane
            for j in range(1, 5):
                sel = vecs[j] > best
                best = jnp.where(sel, vecs[j], best)
                bc = jnp.where(sel, lane + L * j, bc)
            rm = jnp.max(best)
            _sst(scores_loc, loff + r, rm)
            _sst(lb64, r, jnp.min(jnp.where(best == rm, bc, 127)))
            return 0

        lax.fori_loop(0, RCHUNK, row_body, 0, unroll=2)
        pltpu.sync_copy(lb64, labels_sh.at[pl.ds(_base(k) * L, RCHUNK)])

    negv16 = jnp.full((L,), NEG, jnp.float32)
    for v in range(80):
        scores_loc[pl.ds(L * v, L)] = negv16

    bufs = (cls_a, cls_b)
    sems = (sem_a, sem_b)
    descs = [None, None]
    descs[0] = _issue(0, cls_a, sem_a)
    for k in range(NCHUNK):
        if k + 1 < NCHUNK:
            descs[(k + 1) % 2] = _issue(k + 1, bufs[(k + 1) % 2],
                                        sems[(k + 1) % 2])
        descs[k % 2].wait()
        _process(k, bufs[k % 2])

    def cml(g, _):
        _sst(cm_loc, g, jnp.max(scores_loc[pl.ds(L * g, L)]))
        return 0

    lax.fori_loop(0, 80, cml, 0, unroll=4)
    gbase16 = g_lo * L

    def ltk(t, _):
        cvs = [cm_loc[pl.ds(L * v, L)] for v in range(5)]
        m = jnp.max(_vecmax5(cvs))
        g = jnp.int32(1 << 20)
        for v in range(5):
            eq = cvs[v] == m
            cnt = _scal(plsc.all_reduce_population_count(eq))
            ff = _scal(plsc.all_reduce_ffs(eq))
            g = jnp.minimum(g, jnp.where(cnt > 0, L * v + ff, 1 << 20))
        svec = scores_loc[pl.ds(L * g, L)]
        lfv = plsc.all_reduce_ffs(svec == m)
        winl = lane == lfv
        _sst(loc_sc, t, m)
        plsc.store_scatter(loc_idx, [jnp.full((L,), t, jnp.int32)],
                           gbase16 + L * g + lane, mask=winl)
        svec2 = jnp.where(winl, NEG, svec)
        scores_loc[pl.ds(L * g, L)] = svec2
        _sst(cm_loc, g, jnp.max(svec2))
        return 0

    loc_sc[pl.ds(288, L)] = negv16
    lax.fori_loop(0, MAXDET, ltk, 0)
    pltpu.sync_copy(loc_sc, locsc_sh.at[pl.ds(TOPP * s, TOPP)])
    pltpu.sync_copy(loc_idx, locidx_sh.at[pl.ds(TOPP * s, TOPP)])

    plsc.subcore_barrier()

    @pl.when(s == 0)
    def _topk():
        pltpu.sync_copy(locsc_sh, msc)
        pltpu.sync_copy(locidx_sh, midx)

        for v in range(18, 20):
            top_sc[pl.ds(L * v, L)] = jnp.full((L,), NEG, jnp.float32)
        zi = jnp.zeros((L,), jnp.int32)
        for v in range(19):
            top_idx[pl.ds(L * v, L)] = zi

        pos0 = jnp.zeros((L,), jnp.int32)
        heads0 = plsc.load_gather(msc, [lane * TOPP])
        hidx0 = plsc.load_gather(midx, [lane * TOPP])

        def mg_body(t, carry):
            pos, heads, hidx = carry
            m = jnp.max(heads)
            win = lane == plsc.all_reduce_ffs(heads == m)
            tt = jnp.full((L,), t, jnp.int32)
            plsc.store_scatter(top_idx, [tt], hidx, mask=win)
            plsc.store_scatter(top_sc, [tt], heads, mask=win)
            pos = jnp.where(win, pos + 1, pos)
            addr = lane * TOPP + pos
            heads = jnp.where(win, plsc.load_gather(msc, [addr], mask=win),
                              heads)
            hidx = jnp.where(win, plsc.load_gather(midx, [addr], mask=win),
                             hidx)
            return (pos, heads, hidx)

        lax.fori_loop(0, MAXDET, mg_body, (pos0, heads0, hidx0))

        for q in range(4):
            pltpu.async_copy(
                boxt_hbm.at[pl.ds(c * (4 * N) + q * (4 * QN), 4 * QN)],
                box_buf, sem_c).wait()
            for v in range(19):
                idxv = top_idx[pl.ds(L * v, L)]
                inq = (idxv >= q * QN) & (idxv < (q + 1) * QN)
                loc = jnp.clip(idxv - q * QN, 0, QN - 1) * 4
                for k in range(4):
                    vals = plsc.load_gather(box_buf, [loc + k])
                    cur = fb_soa[pl.ds(TOPP * k + L * v, L)]
                    fb_soa[pl.ds(TOPP * k + L * v, L)] = (
                        jnp.where(inq, vals, cur))

        acc = jnp.zeros((L,), jnp.int32)
        for v in range(19):
            vec = top_sc[pl.ds(L * v, L)]
            acc = acc + jnp.where(vec > SCORE_TH, 1, 0).astype(jnp.int32)
        nv_smem[0] = jnp.sum(acc)

        pltpu.sync_copy(fb_soa, fb_sh)

    plsc.subcore_barrier()

    @pl.when((s > 0) & (s <= 13))
    def _iou():
        pltpu.sync_copy(fb_sh, fb_soa)
        r0 = jnp.minimum((s - 1) * 24, TOPP - 24)

        def iou_row(rr, _):
            i = r0 + rr
            ax1 = _sldv(fb_soa, i)
            ay1 = _sldv(fb_soa, TOPP + i)
            ax2 = _sldv(fb_soa, 2 * TOPP + i)
            ay2 = _sldv(fb_soa, 3 * TOPP + i)
            area_a = (ax2 - ax1) * (ay2 - ay1)
            for v in range(19):
                bx1 = fb_soa[pl.ds(L * v, L)]
                by1 = fb_soa[pl.ds(TOPP + L * v, L)]
                bx2 = fb_soa[pl.ds(2 * TOPP + L * v, L)]
                by2 = fb_soa[pl.ds(3 * TOPP + L * v, L)]
                ltx = jnp.maximum(ax1, bx1)
                lty = jnp.maximum(ay1, by1)
                rbx = jnp.minimum(ax2, bx2)
                rby = jnp.minimum(ay2, by2)
                iw = jnp.maximum(rbx - ltx, 0.0)
                ih = jnp.maximum(rby - lty, 0.0)
                area_i = iw * ih
                area_b = (bx2 - bx1) * (by2 - by1)
                area_u = jnp.maximum(area_a + area_b - area_i, 1e-07)
                iou_loc[pl.ds(TOPP * rr + L * v, L)] = area_i / area_u
            return 0

        lax.fori_loop(0, 24, iou_row, 0)
        pltpu.sync_copy(iou_loc, iou_sh.at[pl.ds(r0 * TOPP, 24 * TOPP)])

    @pl.when(s == 0)
    def _labels():
        pltpu.sync_copy(labels_sh, labels_f)
        for v in range(19):
            idxv = top_idx[pl.ds(L * v, L)]
            labels_v[pl.ds(L * v, L)] = plsc.load_gather(labels_f, [idxv])

    plsc.subcore_barrier()

    @pl.when(s == 0)
    def _nms():
        nv = nv_smem[0]
        for v in range(19):
            col = lane + L * v
            alive[pl.ds(L * v, L)] = jnp.where(col < nv, 1, 0).astype(jnp.int32)
        negv = jnp.full((L,), -1.0, jnp.float32)
        negi = jnp.full((L,), -1, jnp.int32)
        for v in range(75):
            stage_b[pl.ds(L * v, L)] = negv
        for v in range(19):
            stage_s[pl.ds(L * v, L)] = negv
            stage_l[pl.ds(L * v, L)] = negi

        cnt = jnp.int32(0)
        for h in range(4):
            pltpu.sync_copy(iou_sh.at[pl.ds(NHALF * h * TOPP, NHALF * TOPP)],
                            iou_half)
            hi = jnp.minimum(nv, NHALF * (h + 1))

            def nms_i(i, cnt):
                def keep_fn(cc):
                    rbase = (i - NHALF * h) * TOPP

                    for v in range(19):
                        iouv = iou_half[pl.ds(rbase + L * v, L)]
                        al = alive[pl.ds(L * v, L)]
                        col = lane + L * v
                        kill = (col > i) & (iouv >= NMS_TH)
                        alive[pl.ds(L * v, L)] = jnp.where(kill, 0, al)
                    _sstv(stage_b, 4 * cc + 0, _sldv(fb_soa, i))
                    _sstv(stage_b, 4 * cc + 1, _sldv(fb_soa, TOPP + i))
                    _sstv(stage_b, 4 * cc + 2, _sldv(fb_soa, 2 * TOPP + i))
                    _sstv(stage_b, 4 * cc + 3, _sldv(fb_soa, 3 * TOPP + i))
                    _sstv(stage_s, cc, _sldv(top_sc, i))
                    _sstv(stage_l, cc, _sldv(labels_v, i))
                    return cc + 1

                return lax.cond(_sldv(alive, i)[0] > 0, keep_fn,
                                lambda cc: cc, cnt)

            cnt = lax.fori_loop(NHALF * h, hi, nms_i, cnt)

        pltpu.sync_copy(stage_b, ob_hbm.at[pl.ds(c * MAXDET * 4, MAXDET * 4)])
        pltpu.sync_copy(stage_s, os_hbm.at[pl.ds(c * TOPP, TOPP)])
        pltpu.sync_copy(stage_l, ol_hbm.at[pl.ds(c * TOPP, TOPP)])


@jax.jit
def kernel(boxes, classification):
    boxt = boxes.reshape(NB * N * 4)
    cls2 = classification.reshape(NB * N, C)
    f32 = jnp.float32
    i32 = jnp.int32
    fd = pl.kernel(
        _fd_body,
        out_type=(
            jax.ShapeDtypeStruct((NB * MAXDET * 4,), f32),
            jax.ShapeDtypeStruct((NB * TOPP,), f32),
            jax.ShapeDtypeStruct((NB * TOPP,), i32),
        ),
        mesh=plsc.VectorSubcoreMesh(core_axis_name="c", subcore_axis_name="s"),
        compiler_params=pltpu.CompilerParams(needs_layout_passes=False),
        scratch_types=[
            pltpu.VMEM((RCHUNK, C), f32),
            pltpu.VMEM((RCHUNK, C), f32),
            pltpu.VMEM((RCHUNK,), i32),
            pltpu.VMEM((1280,), f32),
            pltpu.VMEM((80,), f32),
            pltpu.VMEM((TOPP,), f32),
            pltpu.VMEM((TOPP,), i32),
            pltpu.VMEM((NS * TOPP,), f32),
            pltpu.VMEM((NS * TOPP,), i32),
            pltpu.VMEM((N,), i32),
            pltpu.VMEM((TOPP,), i32),
            pltpu.VMEM((320,), f32),
            pltpu.VMEM((TOPP,), i32),
            pltpu.VMEM((4 * TOPP,), f32),
            pltpu.VMEM((4 * QN,), f32),
            pltpu.VMEM((24 * TOPP,), f32),
            pltpu.VMEM((NHALF * TOPP,), f32),
            pltpu.VMEM((TOPP,), i32),
            pltpu.VMEM((MAXDET * 4,), f32),
            pltpu.VMEM((TOPP,), f32),
            pltpu.VMEM((TOPP,), i32),
            pltpu.SMEM((1,), i32),
            pltpu.VMEM_SHARED((NS * TOPP,), f32),
            pltpu.VMEM_SHARED((NS * TOPP,), i32),
            pltpu.VMEM_SHARED((N,), i32),
            pltpu.VMEM_SHARED((4 * TOPP,), f32),
            pltpu.VMEM_SHARED((TOPP * TOPP,), f32),
            pltpu.SemaphoreType.DMA,
            pltpu.SemaphoreType.DMA,
            pltpu.SemaphoreType.DMA,
        ],
    )
    ob, os_, ol = fd(boxt, cls2)
    return (ob.reshape(NB, MAXDET, 4),
            os_.reshape(NB, TOPP)[:, :MAXDET],
            ol.reshape(NB, TOPP)[:, :MAXDET])


# speedup vs baseline: 1.5287x; 1.5287x over previous
import jax
import jax.numpy as jnp
from jax import lax
from jax.experimental import pallas as pl
from jax.experimental.pallas import tpu as pltpu
from jax.experimental.pallas import tpu_sc as plsc

SCORE_TH = 0.05
NMS_TH = 0.5
MAXDET = 300
N = 20000
C = 80
NB = 2
NS = 16
L = 16
NBLK = 157
NFULL = 156
NCHUNK = 10
NEG = -3.0e38
TOPP = 304
QN = 5000
NHALF = 76


def _vecmax5(vecs):
    m01 = jnp.maximum(vecs[0], vecs[1])
    m23 = jnp.maximum(vecs[2], vecs[3])
    return jnp.maximum(jnp.maximum(m01, m23), vecs[4])


def _fd_body(boxt_hbm, cls_hbm, ctail_hbm, btail_hbm, ob_hbm, os_hbm, ol_hbm,
             ct_a, ct_tail, btail_buf, lb_chunk, scores_loc, cm_loc,
             loc_sc, loc_idx,
             msc, midx, labels_f,
             top_idx, top_sc, labels_v, fb_soa, box_buf,
             iou_loc, iou_half, alive, stage_b, stage_s, stage_l, nv_smem,
             locsc_sh, locidx_sh, labels_sh, fb_sh, iou_sh,
             sem_a, sem_b, sem_c):
    c = lax.axis_index("c")
    s = lax.axis_index("s")
    lane = lax.iota(jnp.int32, L)
    lane0 = lane == 0

    def _sst(ref, idx, val):
        plsc.store_scatter(ref, [jnp.full((L,), idx, jnp.int32)],
                           jnp.full((L,), val), mask=lane0)

    def _sstv(ref, idx, vec):
        plsc.store_scatter(ref, [jnp.full((L,), idx, jnp.int32)], vec,
                           mask=lane0)

    def _sldv(ref, idx):
        return plsc.load_gather(ref, [jnp.full((L,), idx, jnp.int32)])

    def _scal(x):
        return x[0] if getattr(x, "ndim", 0) else x

    blo = (NBLK * s) // NS
    bhi = (NBLK * (s + 1)) // NS
    fbhi = jnp.minimum(bhi, NFULL)
    b_lo = blo * 128

    negv16 = jnp.full((L,), NEG, jnp.float32)
    for v in range(80):
        scores_loc[pl.ds(L * v, L)] = negv16

    def _groups(buf, lbuf, ngroups, col0):
        def grp(g, _):
            best = buf[0, pl.ds(L * g, L)]
            bc = jnp.zeros((L,), jnp.int32)
            for j in range(1, C):
                v = buf[j, pl.ds(L * g, L)]
                sel = v > best
                best = jnp.where(sel, v, best)
                bc = jnp.where(sel, j, bc)
            scores_loc[pl.ds(col0 - b_lo + L * g, L)] = best
            lbuf[pl.ds(L * g, L)] = bc
            return 0

        lax.fori_loop(0, ngroups, grp, 0)

    def chunk_body(k, _):
        base_blk = jnp.minimum(blo + k, fbhi - 1)
        col0 = base_blk * 128
        pltpu.sync_copy(
            cls_hbm.at[pl.ds(c * C, C), pl.ds(col0, 128)], ct_a)
        _groups(ct_a, lb_chunk, 8, col0)
        pltpu.sync_copy(lb_chunk, labels_sh.at[pl.ds(col0, 128)])
        return 0

    lax.fori_loop(0, NCHUNK, chunk_body, 0)

    @pl.when(s == NS - 1)
    def _tail():
        col0 = NFULL * 128
        pltpu.sync_copy(ctail_hbm, ct_tail)

        def trow(r, _):
            vecs = [ct_tail[c * 32 + r, pl.ds(L * j, L)] for j in range(5)]
            best = vecs[0]
            bc = lane
            for j in range(1, 5):
                sel = vecs[j] > best
                best = jnp.where(sel, vecs[j], best)
                bc = jnp.where(sel, lane + L * j, bc)
            rm = jnp.max(best)
            _sst(scores_loc, col0 - b_lo + r, rm)
            _sst(lb_chunk, r, jnp.min(jnp.where(best == rm, bc, 127)))
            return 0

        lax.fori_loop(0, 32, trow, 0)
        pltpu.sync_copy(lb_chunk.at[pl.ds(0, 32)],
                        labels_sh.at[pl.ds(col0, 32)])

    def cml(g, _):
        _sst(cm_loc, g, jnp.max(scores_loc[pl.ds(L * g, L)]))
        return 0

    lax.fori_loop(0, 80, cml, 0, unroll=4)

    def ltk(t, _):
        cvs = [cm_loc[pl.ds(L * v, L)] for v in range(5)]
        m = jnp.max(_vecmax5(cvs))
        g = jnp.int32(1 << 20)
        for v in range(5):
            eq = cvs[v] == m
            cnt = _scal(plsc.all_reduce_population_count(eq))
            ff = _scal(plsc.all_reduce_ffs(eq))
            g = jnp.minimum(g, jnp.where(cnt > 0, L * v + ff, 1 << 20))
        svec = scores_loc[pl.ds(L * g, L)]
        lfv = plsc.all_reduce_ffs(svec == m)
        winl = lane == lfv
        _sst(loc_sc, t, m)
        plsc.store_scatter(loc_idx, [jnp.full((L,), t, jnp.int32)],
                           b_lo + L * g + lane, mask=winl)
        svec2 = jnp.where(winl, NEG, svec)
        scores_loc[pl.ds(L * g, L)] = svec2
        _sst(cm_loc, g, jnp.max(svec2))
        return 0

    loc_sc[pl.ds(288, L)] = negv16
    lax.fori_loop(0, MAXDET, ltk, 0)
    pltpu.sync_copy(loc_sc, locsc_sh.at[pl.ds(TOPP * s, TOPP)])
    pltpu.sync_copy(loc_idx, locidx_sh.at[pl.ds(TOPP * s, TOPP)])

    plsc.subcore_barrier()

    @pl.when(s == 0)
    def _topk():
        pltpu.sync_copy(locsc_sh, msc)
        pltpu.sync_copy(locidx_sh, midx)

        for v in range(18, 20):
            top_sc[pl.ds(L * v, L)] = jnp.full((L,), NEG, jnp.float32)
        zi = jnp.zeros((L,), jnp.int32)
        for v in range(19):
            top_idx[pl.ds(L * v, L)] = zi

        pos0 = jnp.zeros((L,), jnp.int32)
        heads0 = plsc.load_gather(msc, [lane * TOPP])
        hidx0 = plsc.load_gather(midx, [lane * TOPP])

        def mg_body(t, carry):
            pos, heads, hidx = carry
            m = jnp.max(heads)
            win = lane == plsc.all_reduce_ffs(heads == m)
            tt = jnp.full((L,), t, jnp.int32)
            plsc.store_scatter(top_idx, [tt], hidx, mask=win)
            plsc.store_scatter(top_sc, [tt], heads, mask=win)
            pos = jnp.where(win, pos + 1, pos)
            addr = lane * TOPP + pos
            heads = jnp.where(win, plsc.load_gather(msc, [addr], mask=win),
                              heads)
            hidx = jnp.where(win, plsc.load_gather(midx, [addr], mask=win),
                             hidx)
            return (pos, heads, hidx)

        lax.fori_loop(0, MAXDET, mg_body, (pos0, heads0, hidx0))

        rounds = [(2560 * q, 2560) for q in range(7)] + [(17920, 2048)]
        for col0, w in rounds:
            pltpu.async_copy(boxt_hbm.at[c, :, pl.ds(col0, w)],
                             box_buf.at[:, pl.ds(0, w)], sem_c).wait()
            for v in range(19):
                idxv = top_idx[pl.ds(L * v, L)]
                inq = (idxv >= col0) & (idxv < col0 + w)
                loc = jnp.clip(idxv - col0, 0, w - 1)
                for k in range(4):
                    vals = plsc.load_gather(
                        box_buf, [jnp.full((L,), k, jnp.int32), loc])
                    cur = fb_soa[pl.ds(TOPP * k + L * v, L)]
                    fb_soa[pl.ds(TOPP * k + L * v, L)] = (
                        jnp.where(inq, vals, cur))
        pltpu.sync_copy(btail_hbm, btail_buf)
        for v in range(19):
            idxv = top_idx[pl.ds(L * v, L)]
            inq = idxv >= NFULL * 128
            loc = jnp.clip(idxv - NFULL * 128, 0, 31)
            for k in range(4):
                vals = plsc.load_gather(
                    btail_buf, [jnp.full((L,), c * 4 + k, jnp.int32), loc])
                cur = fb_soa[pl.ds(TOPP * k + L * v, L)]
                fb_soa[pl.ds(TOPP * k + L * v, L)] = (
                    jnp.where(inq, vals, cur))

        acc = jnp.zeros((L,), jnp.int32)
        for v in range(19):
            vec = top_sc[pl.ds(L * v, L)]
            acc = acc + jnp.where(vec > SCORE_TH, 1, 0).astype(jnp.int32)
        nv_smem[0] = jnp.sum(acc)

        pltpu.sync_copy(fb_soa, fb_sh)

    plsc.subcore_barrier()

    @pl.when((s > 0) & (s <= 13))
    def _iou():
        pltpu.sync_copy(fb_sh, fb_soa)
        r0 = jnp.minimum((s - 1) * 24, TOPP - 24)

        def iou_row(rr, _):
            i = r0 + rr
            ax1 = _sldv(fb_soa, i)
            ay1 = _sldv(fb_soa, TOPP + i)
            ax2 = _sldv(fb_soa, 2 * TOPP + i)
            ay2 = _sldv(fb_soa, 3 * TOPP + i)
            area_a = (ax2 - ax1) * (ay2 - ay1)
            for v in range(19):
                bx1 = fb_soa[pl.ds(L * v, L)]
                by1 = fb_soa[pl.ds(TOPP + L * v, L)]
                bx2 = fb_soa[pl.ds(2 * TOPP + L * v, L)]
                by2 = fb_soa[pl.ds(3 * TOPP + L * v, L)]
                ltx = jnp.maximum(ax1, bx1)
                lty = jnp.maximum(ay1, by1)
                rbx = jnp.minimum(ax2, bx2)
                rby = jnp.minimum(ay2, by2)
                iw = jnp.maximum(rbx - ltx, 0.0)
                ih = jnp.maximum(rby - lty, 0.0)
                area_i = iw * ih
                area_b = (bx2 - bx1) * (by2 - by1)
                area_u = jnp.maximum(area_a + area_b - area_i, 1e-07)
                iou_loc[pl.ds(TOPP * rr + L * v, L)] = area_i / area_u
            return 0

        lax.fori_loop(0, 24, iou_row, 0)
        pltpu.sync_copy(iou_loc, iou_sh.at[pl.ds(r0 * TOPP, 24 * TOPP)])

    @pl.when(s == 0)
    def _labels():
        pltpu.sync_copy(labels_sh, labels_f)
        for v in range(19):
            idxv = top_idx[pl.ds(L * v, L)]
            labels_v[pl.ds(L * v, L)] = plsc.load_gather(labels_f, [idxv])

    plsc.subcore_barrier()

    @pl.when(s == 0)
    def _nms():
        nv = nv_smem[0]
        for v in range(19):
            col = lane + L * v
            alive[pl.ds(L * v, L)] = jnp.where(col < nv, 1, 0).astype(jnp.int32)
        negv = jnp.full((L,), -1.0, jnp.float32)
        negi = jnp.full((L,), -1, jnp.int32)
        for v in range(75):
            stage_b[pl.ds(L * v, L)] = negv
        for v in range(19):
            stage_s[pl.ds(L * v, L)] = negv
            stage_l[pl.ds(L * v, L)] = negi

        cnt = jnp.int32(0)
        for h in range(4):
            pltpu.sync_copy(iou_sh.at[pl.ds(NHALF * h * TOPP, NHALF * TOPP)],
                            iou_half)
            hi = jnp.minimum(nv, NHALF * (h + 1))

            def nms_i(i, cnt):
                def keep_fn(cc):
                    rbase = (i - NHALF * h) * TOPP

                    for v in range(19):
                        iouv = iou_half[pl.ds(rbase + L * v, L)]
                        al = alive[pl.ds(L * v, L)]
                        col = lane + L * v
                        kill = (col > i) & (iouv >= NMS_TH)
                        alive[pl.ds(L * v, L)] = jnp.where(kill, 0, al)
                    _sstv(stage_b, 4 * cc + 0, _sldv(fb_soa, i))
                    _sstv(stage_b, 4 * cc + 1, _sldv(fb_soa, TOPP + i))
                    _sstv(stage_b, 4 * cc + 2, _sldv(fb_soa, 2 * TOPP + i))
                    _sstv(stage_b, 4 * cc + 3, _sldv(fb_soa, 3 * TOPP + i))
                    _sstv(stage_s, cc, _sldv(top_sc, i))
                    _sstv(stage_l, cc, _sldv(labels_v, i))
                    return cc + 1

                return lax.cond(_sldv(alive, i)[0] > 0, keep_fn,
                                lambda cc: cc, cnt)

            cnt = lax.fori_loop(NHALF * h, hi, nms_i, cnt)

        pltpu.sync_copy(stage_b, ob_hbm.at[pl.ds(c * MAXDET * 4, MAXDET * 4)])
        pltpu.sync_copy(stage_s, os_hbm.at[pl.ds(c * TOPP, TOPP)])
        pltpu.sync_copy(stage_l, ol_hbm.at[pl.ds(c * TOPP, TOPP)])


@jax.jit
def kernel(boxes, classification):
    boxt = boxes.transpose(0, 2, 1)
    clsT = classification.transpose(0, 2, 1).reshape(NB * C, N)
    ctail = classification[:, NFULL * 128:, :].reshape(2 * 32, C)
    btail = boxes[:, NFULL * 128:, :].transpose(0, 2, 1).reshape(8, 32)
    f32 = jnp.float32
    i32 = jnp.int32
    fd = pl.kernel(
        _fd_body,
        out_type=(
            jax.ShapeDtypeStruct((NB * MAXDET * 4,), f32),
            jax.ShapeDtypeStruct((NB * TOPP,), f32),
            jax.ShapeDtypeStruct((NB * TOPP,), i32),
        ),
        mesh=plsc.VectorSubcoreMesh(core_axis_name="c", subcore_axis_name="s"),
        compiler_params=pltpu.CompilerParams(needs_layout_passes=False),
        scratch_types=[
            pltpu.VMEM((C, 128), f32),
            pltpu.VMEM((2 * 32, C), f32),
            pltpu.VMEM((8, 32), f32),
            pltpu.VMEM((128,), i32),
            pltpu.VMEM((1280,), f32),
            pltpu.VMEM((80,), f32),
            pltpu.VMEM((TOPP,), f32),
            pltpu.VMEM((TOPP,), i32),
            pltpu.VMEM((NS * TOPP,), f32),
            pltpu.VMEM((NS * TOPP,), i32),
            pltpu.VMEM((N,), i32),
            pltpu.VMEM((TOPP,), i32),
            pltpu.VMEM((320,), f32),
            pltpu.VMEM((TOPP,), i32),
            pltpu.VMEM((4 * TOPP,), f32),
            pltpu.VMEM((4, 2560), f32),
            pltpu.VMEM((24 * TOPP,), f32),
            pltpu.VMEM((NHALF * TOPP,), f32),
            pltpu.VMEM((TOPP,), i32),
            pltpu.VMEM((MAXDET * 4,), f32),
            pltpu.VMEM((TOPP,), f32),
            pltpu.VMEM((TOPP,), i32),
            pltpu.SMEM((1,), i32),
            pltpu.VMEM_SHARED((NS * TOPP,), f32),
            pltpu.VMEM_SHARED((NS * TOPP,), i32),
            pltpu.VMEM_SHARED((N,), i32),
            pltpu.VMEM_SHARED((4 * TOPP,), f32),
            pltpu.VMEM_SHARED((TOPP * TOPP,), f32),
            pltpu.SemaphoreType.DMA,
            pltpu.SemaphoreType.DMA,
            pltpu.SemaphoreType.DMA,
        ],
    )
    ob, os_, ol = fd(boxt, clsT, ctail, btail)
    return (ob.reshape(NB, MAXDET, 4),
            os_.reshape(NB, TOPP)[:, :MAXDET],
            ol.reshape(NB, TOPP)[:, :MAXDET])


# speedup vs baseline: 1.8887x; 1.2355x over previous
import jax
import jax.numpy as jnp
from jax import lax
from jax.experimental import pallas as pl
from jax.experimental.pallas import tpu as pltpu
from jax.experimental.pallas import tpu_sc as plsc

SCORE_TH = 0.05
NMS_TH = 0.5
MAXDET = 300
N = 20000
C = 80
NB = 2
NS = 16
L = 16
NBLK = 157
NFULL = 156
NCHUNK = 10
NEG = -3.0e38
K1 = 64
TOPP = 304
QN = 5000
NHALF = 76


def _vecmax5(vecs):
    m01 = jnp.maximum(vecs[0], vecs[1])
    m23 = jnp.maximum(vecs[2], vecs[3])
    return jnp.maximum(jnp.maximum(m01, m23), vecs[4])


def _fd_body(boxt_hbm, cls_hbm, ctail_hbm, btail_hbm, ob_hbm, os_hbm, ol_hbm,
             ct_a, ct_b, ct_tail, btail_buf, lb_chunk, scores_loc, cm_loc,
             loc_sc, loc_idx,
             msc, midx, labels_f,
             top_idx, top_sc, labels_v, fb_soa, box_buf,
             iou_loc, iou_half, alive, stage_b, stage_s, stage_l, fbuf,
             nv_smem, locsc_sh, locidx_sh, labels_sh, fb_sh, iou_sh,
             flag_sh, sem_a, sem_b, sem_c):
    c = lax.axis_index("c")
    s = lax.axis_index("s")
    lane = lax.iota(jnp.int32, L)
    lane0 = lane == 0

    def _sst(ref, idx, val):
        plsc.store_scatter(ref, [jnp.full((L,), idx, jnp.int32)],
                           jnp.full((L,), val), mask=lane0)

    def _sstv(ref, idx, vec):
        plsc.store_scatter(ref, [jnp.full((L,), idx, jnp.int32)], vec,
                           mask=lane0)

    def _sldv(ref, idx):
        return plsc.load_gather(ref, [jnp.full((L,), idx, jnp.int32)])

    def _scal(x):
        return x[0] if getattr(x, "ndim", 0) else x

    blo = (NBLK * s) // NS
    bhi = (NBLK * (s + 1)) // NS
    fbhi = jnp.minimum(bhi, NFULL)
    b_lo = blo * 128

    negv16 = jnp.full((L,), NEG, jnp.float32)
    for v in range(80):
        scores_loc[pl.ds(L * v, L)] = negv16

    def _groups(buf, lbuf, ngroups, col0):
        def grp(g, _):
            best = buf[0, pl.ds(L * g, L)]
            bc = jnp.zeros((L,), jnp.int32)
            for j in range(1, C):
                v = buf[j, pl.ds(L * g, L)]
                sel = v > best
                best = jnp.where(sel, v, best)
                bc = jnp.where(sel, j, bc)
            scores_loc[pl.ds(col0 - b_lo + L * g, L)] = best
            lbuf[pl.ds(L * g, L)] = bc
            return 0

        lax.fori_loop(0, ngroups, grp, 0)

    def _cissue(blk_k, buf, sem):
        base_blk = jnp.minimum(blo + blk_k, fbhi - 1)
        pltpu.async_copy(
            cls_hbm.at[pl.ds(c * C, C), pl.ds(base_blk * 128, 128)],
            buf, sem)

    _cissue(0, ct_a, sem_a)
    _cissue(1, ct_b, sem_b)

    def chunk2(k2, _):
        for b, (buf, sem) in enumerate(((ct_a, sem_a), (ct_b, sem_b))):
            k = 2 * k2 + b
            pltpu.make_async_copy(
                cls_hbm.at[pl.ds(0, C), pl.ds(0, 128)], buf, sem).wait()
            base_blk = jnp.minimum(blo + k, fbhi - 1)
            col0 = base_blk * 128
            _groups(buf, lb_chunk, 8, col0)
            pltpu.sync_copy(lb_chunk, labels_sh.at[pl.ds(col0, 128)])

            @pl.when(k + 2 < NCHUNK)
            def _():
                _cissue(k + 2, buf, sem)

        return 0

    lax.fori_loop(0, NCHUNK // 2, chunk2, 0)

    @pl.when(s == NS - 1)
    def _tail():
        col0 = NFULL * 128
        pltpu.sync_copy(ctail_hbm, ct_tail)

        def trow(r, _):
            vecs = [ct_tail[c * 32 + r, pl.ds(L * j, L)] for j in range(5)]
            best = vecs[0]
            bc = lane
            for j in range(1, 5):
                sel = vecs[j] > best
                best = jnp.where(sel, vecs[j], best)
                bc = jnp.where(sel, lane + L * j, bc)
            rm = jnp.max(best)
            _sst(scores_loc, col0 - b_lo + r, rm)
            _sst(lb_chunk, r, jnp.min(jnp.where(best == rm, bc, 127)))
            return 0

        lax.fori_loop(0, 32, trow, 0)
        pltpu.sync_copy(lb_chunk.at[pl.ds(0, 32)],
                        labels_sh.at[pl.ds(col0, 32)])

    def cml(g, _):
        _sst(cm_loc, g, jnp.max(scores_loc[pl.ds(L * g, L)]))
        return 0

    lax.fori_loop(0, 80, cml, 0, unroll=4)

    def ltk(t, _):
        cvs = [cm_loc[pl.ds(L * v, L)] for v in range(5)]
        m = jnp.max(_vecmax5(cvs))
        g = jnp.int32(1 << 20)
        for v in range(5):
            eq = cvs[v] == m
            cnt = _scal(plsc.all_reduce_population_count(eq))
            ff = _scal(plsc.all_reduce_ffs(eq))
            g = jnp.minimum(g, jnp.where(cnt > 0, L * v + ff, 1 << 20))
        svec = scores_loc[pl.ds(L * g, L)]
        lfv = plsc.all_reduce_ffs(svec == m)
        winl = lane == lfv
        _sst(loc_sc, t, m)
        plsc.store_scatter(loc_idx, [jnp.full((L,), t, jnp.int32)],
                           b_lo + L * g + lane, mask=winl)
        svec2 = jnp.where(winl, NEG, svec)
        scores_loc[pl.ds(L * g, L)] = svec2
        _sst(cm_loc, g, jnp.max(svec2))
        return 0

    for v in range(19):
        loc_sc[pl.ds(L * v, L)] = negv16
    lax.fori_loop(0, K1, ltk, 0)
    pltpu.sync_copy(loc_sc, locsc_sh.at[pl.ds(TOPP * s, TOPP)])
    pltpu.sync_copy(loc_idx, locidx_sh.at[pl.ds(TOPP * s, TOPP)])

    plsc.subcore_barrier()

    def _do_merge():
        pos0 = jnp.zeros((L,), jnp.int32)
        heads0 = plsc.load_gather(msc, [lane * TOPP])
        hidx0 = plsc.load_gather(midx, [lane * TOPP])

        def mg_body(t, carry):
            pos, heads, hidx = carry
            m = jnp.max(heads)
            win = lane == plsc.all_reduce_ffs(heads == m)
            tt = jnp.full((L,), t, jnp.int32)
            plsc.store_scatter(top_idx, [tt], hidx, mask=win)
            plsc.store_scatter(top_sc, [tt], heads, mask=win)
            pos = jnp.where(win, pos + 1, pos)
            addr = lane * TOPP + pos
            heads = jnp.where(win, plsc.load_gather(msc, [addr], mask=win),
                              heads)
            hidx = jnp.where(win, plsc.load_gather(midx, [addr], mask=win),
                             hidx)
            return (pos, heads, hidx)

        pos, _, _ = lax.fori_loop(0, MAXDET, mg_body,
                                  (pos0, heads0, hidx0))
        return pos

    @pl.when(s == 0)
    def _topk1():
        pltpu.sync_copy(locsc_sh, msc)
        pltpu.sync_copy(locidx_sh, midx)

        for v in range(18, 20):
            top_sc[pl.ds(L * v, L)] = jnp.full((L,), NEG, jnp.float32)
        zi = jnp.zeros((L,), jnp.int32)
        for v in range(19):
            top_idx[pl.ds(L * v, L)] = zi

        pos = _do_merge()
        drained = jnp.max(jnp.where(pos >= K1, 1, 0).astype(jnp.int32))
        fbuf[pl.ds(0, L)] = jnp.zeros((L,), jnp.int32) + drained
        pltpu.sync_copy(fbuf, flag_sh)

    plsc.subcore_barrier()
    pltpu.sync_copy(flag_sh, fbuf)
    bad = fbuf[pl.ds(0, L)][0]

    @pl.when(bad > 0)
    def _extend():
        lax.fori_loop(K1, MAXDET, ltk, 0)
        pltpu.sync_copy(loc_sc, locsc_sh.at[pl.ds(TOPP * s, TOPP)])
        pltpu.sync_copy(loc_idx, locidx_sh.at[pl.ds(TOPP * s, TOPP)])

    plsc.subcore_barrier()

    @pl.when(s == 0)
    def _topk2():
        @pl.when(bad > 0)
        def _remerge():
            pltpu.sync_copy(locsc_sh, msc)
            pltpu.sync_copy(locidx_sh, midx)
            _do_merge()

        rounds = [(2560 * q, 2560) for q in range(7)] + [(17920, 2048)]
        for col0, w in rounds:
            pltpu.async_copy(boxt_hbm.at[c, :, pl.ds(col0, w)],
                             box_buf.at[:, pl.ds(0, w)], sem_c).wait()
            for v in range(19):
                idxv = top_idx[pl.ds(L * v, L)]
                inq = (idxv >= col0) & (idxv < col0 + w)
                loc = jnp.clip(idxv - col0, 0, w - 1)
                for k in range(4):
                    vals = plsc.load_gather(
                        box_buf, [jnp.full((L,), k, jnp.int32), loc])
                    cur = fb_soa[pl.ds(TOPP * k + L * v, L)]
                    fb_soa[pl.ds(TOPP * k + L * v, L)] = (
                        jnp.where(inq, vals, cur))
        pltpu.sync_copy(btail_hbm, btail_buf)
        for v in range(19):
            idxv = top_idx[pl.ds(L * v, L)]
            inq = idxv >= NFULL * 128
            loc = jnp.clip(idxv - NFULL * 128, 0, 31)
            for k in range(4):
                vals = plsc.load_gather(
                    btail_buf, [jnp.full((L,), c * 4 + k, jnp.int32), loc])
                cur = fb_soa[pl.ds(TOPP * k + L * v, L)]
                fb_soa[pl.ds(TOPP * k + L * v, L)] = (
                    jnp.where(inq, vals, cur))

        acc = jnp.zeros((L,), jnp.int32)
        for v in range(19):
            vec = top_sc[pl.ds(L * v, L)]
            acc = acc + jnp.where(vec > SCORE_TH, 1, 0).astype(jnp.int32)
        nv_smem[0] = jnp.sum(acc)

        pltpu.sync_copy(fb_soa, fb_sh)

    plsc.subcore_barrier()

    @pl.when((s > 0) & (s <= 13))
    def _iou():
        pltpu.sync_copy(fb_sh, fb_soa)
        r0 = jnp.minimum((s - 1) * 24, TOPP - 24)

        def iou_row(rr, _):
            i = r0 + rr
            ax1 = _sldv(fb_soa, i)
            ay1 = _sldv(fb_soa, TOPP + i)
            ax2 = _sldv(fb_soa, 2 * TOPP + i)
            ay2 = _sldv(fb_soa, 3 * TOPP + i)
            area_a = (ax2 - ax1) * (ay2 - ay1)
            for v in range(19):
                bx1 = fb_soa[pl.ds(L * v, L)]
                by1 = fb_soa[pl.ds(TOPP + L * v, L)]
                bx2 = fb_soa[pl.ds(2 * TOPP + L * v, L)]
                by2 = fb_soa[pl.ds(3 * TOPP + L * v, L)]
                ltx = jnp.maximum(ax1, bx1)
                lty = jnp.maximum(ay1, by1)
                rbx = jnp.minimum(ax2, bx2)
                rby = jnp.minimum(ay2, by2)
                iw = jnp.maximum(rbx - ltx, 0.0)
                ih = jnp.maximum(rby - lty, 0.0)
                area_i = iw * ih
                area_b = (bx2 - bx1) * (by2 - by1)
                area_u = jnp.maximum(area_a + area_b - area_i, 1e-07)
                iou_loc[pl.ds(TOPP * rr + L * v, L)] = area_i / area_u
            return 0

        lax.fori_loop(0, 24, iou_row, 0)
        pltpu.sync_copy(iou_loc, iou_sh.at[pl.ds(r0 * TOPP, 24 * TOPP)])

    @pl.when(s == 0)
    def _labels():
        pltpu.sync_copy(labels_sh, labels_f)
        for v in range(19):
            idxv = top_idx[pl.ds(L * v, L)]
            labels_v[pl.ds(L * v, L)] = plsc.load_gather(labels_f, [idxv])

    plsc.subcore_barrier()

    @pl.when(s == 0)
    def _nms():
        nv = nv_smem[0]
        for v in range(19):
            col = lane + L * v
            alive[pl.ds(L * v, L)] = jnp.where(col < nv, 1, 0).astype(jnp.int32)
        negv = jnp.full((L,), -1.0, jnp.float32)
        negi = jnp.full((L,), -1, jnp.int32)
        for v in range(75):
            stage_b[pl.ds(L * v, L)] = negv
        for v in range(19):
            stage_s[pl.ds(L * v, L)] = negv
            stage_l[pl.ds(L * v, L)] = negi

        cnt = jnp.int32(0)
        for h in range(4):
            pltpu.sync_copy(iou_sh.at[pl.ds(NHALF * h * TOPP, NHALF * TOPP)],
                            iou_half)
            hi = jnp.minimum(nv, NHALF * (h + 1))

            def nms_i(i, cnt):
                def keep_fn(cc):
                    rbase = (i - NHALF * h) * TOPP

                    for v in range((NHALF * h) // L, 19):
                        iouv = iou_half[pl.ds(rbase + L * v, L)]
                        al = alive[pl.ds(L * v, L)]
                        col = lane + L * v
                        kill = (col > i) & (iouv >= NMS_TH)
                        alive[pl.ds(L * v, L)] = jnp.where(kill, 0, al)
                    _sstv(stage_b, 4 * cc + 0, _sldv(fb_soa, i))
                    _sstv(stage_b, 4 * cc + 1, _sldv(fb_soa, TOPP + i))
                    _sstv(stage_b, 4 * cc + 2, _sldv(fb_soa, 2 * TOPP + i))
                    _sstv(stage_b, 4 * cc + 3, _sldv(fb_soa, 3 * TOPP + i))
                    _sstv(stage_s, cc, _sldv(top_sc, i))
                    _sstv(stage_l, cc, _sldv(labels_v, i))
                    return cc + 1

                return lax.cond(_sldv(alive, i)[0] > 0, keep_fn,
                                lambda cc: cc, cnt)

            cnt = lax.fori_loop(NHALF * h, hi, nms_i, cnt)

        pltpu.sync_copy(stage_b, ob_hbm.at[pl.ds(c * MAXDET * 4, MAXDET * 4)])
        pltpu.sync_copy(stage_s, os_hbm.at[pl.ds(c * TOPP, TOPP)])
        pltpu.sync_copy(stage_l, ol_hbm.at[pl.ds(c * TOPP, TOPP)])


@jax.jit
def kernel(boxes, classification):
    boxt = boxes.transpose(0, 2, 1)
    clsT = classification.transpose(0, 2, 1).reshape(NB * C, N)
    ctail = classification[:, NFULL * 128:, :].reshape(2 * 32, C)
    btail = boxes[:, NFULL * 128:, :].transpose(0, 2, 1).reshape(8, 32)
    f32 = jnp.float32
    i32 = jnp.int32
    fd = pl.kernel(
        _fd_body,
        out_type=(
            jax.ShapeDtypeStruct((NB * MAXDET * 4,), f32),
            jax.ShapeDtypeStruct((NB * TOPP,), f32),
            jax.ShapeDtypeStruct((NB * TOPP,), i32),
        ),
        mesh=plsc.VectorSubcoreMesh(core_axis_name="c", subcore_axis_name="s"),
        compiler_params=pltpu.CompilerParams(needs_layout_passes=False),
        scratch_types=[
            pltpu.VMEM((C, 128), f32),
            pltpu.VMEM((C, 128), f32),
            pltpu.VMEM((2 * 32, C), f32),
            pltpu.VMEM((8, 32), f32),
            pltpu.VMEM((128,), i32),
            pltpu.VMEM((1280,), f32),
            pltpu.VMEM((80,), f32),
            pltpu.VMEM((TOPP,), f32),
            pltpu.VMEM((TOPP,), i32),
            pltpu.VMEM((NS * TOPP,), f32),
            pltpu.VMEM((NS * TOPP,), i32),
            pltpu.VMEM((N,), i32),
            pltpu.VMEM((TOPP,), i32),
            pltpu.VMEM((320,), f32),
            pltpu.VMEM((TOPP,), i32),
            pltpu.VMEM((4 * TOPP,), f32),
            pltpu.VMEM((4, 2560), f32),
            pltpu.VMEM((24 * TOPP,), f32),
            pltpu.VMEM((NHALF * TOPP,), f32),
            pltpu.VMEM((TOPP,), i32),
            pltpu.VMEM((MAXDET * 4,), f32),
            pltpu.VMEM((TOPP,), f32),
            pltpu.VMEM((TOPP,), i32),
            pltpu.VMEM((L,), i32),
            pltpu.SMEM((1,), i32),
            pltpu.VMEM_SHARED((NS * TOPP,), f32),
            pltpu.VMEM_SHARED((NS * TOPP,), i32),
            pltpu.VMEM_SHARED((N,), i32),
            pltpu.VMEM_SHARED((4 * TOPP,), f32),
            pltpu.VMEM_SHARED((TOPP * TOPP,), f32),
            pltpu.VMEM_SHARED((L,), i32),
            pltpu.SemaphoreType.DMA,
            pltpu.SemaphoreType.DMA,
            pltpu.SemaphoreType.DMA,
        ],
    )
    ob, os_, ol = fd(boxt, clsT, ctail, btail)
    return (ob.reshape(NB, MAXDET, 4),
            os_.reshape(NB, TOPP)[:, :MAXDET],
            ol.reshape(NB, TOPP)[:, :MAXDET])


# speedup vs baseline: 1.9199x; 1.0165x over previous
import jax
import jax.numpy as jnp
from jax import lax
from jax.experimental import pallas as pl
from jax.experimental.pallas import tpu as pltpu
from jax.experimental.pallas import tpu_sc as plsc

SCORE_TH = 0.05
NMS_TH = 0.5
MAXDET = 300
N = 20000
C = 80
NB = 2
NS = 16
L = 16
NBLK = 157
NFULL = 156
NCHUNK = 10
NEG = -3.0e38
K1 = 64
TOPP = 304
NHALF = 76


def _vecmax5(vecs):
    m01 = jnp.maximum(vecs[0], vecs[1])
    m23 = jnp.maximum(vecs[2], vecs[3])
    return jnp.maximum(jnp.maximum(m01, m23), vecs[4])


def _fd_body(boxt_hbm, cls_hbm, ctail_hbm, btail_hbm, ob_hbm, os_hbm, ol_hbm,
             ct_a, ct_b, ct_tail, btail_buf, lb_chunk, scores_loc, cm_loc,
             loc_sc, loc_idx,
             msc, midx, labels_f,
             top_idx, top_sc, labels_v, fb_soa, box_buf,
             iou_loc, iou_half, alive, stage_b, stage_s, stage_l, fbuf,
             nv_smem, locsc_sh, locidx_sh, labels_sh, fb_sh, iou_sh,
             flag_sh, sem_a, sem_b, sem_c):
    c = lax.axis_index("c")
    s = lax.axis_index("s")
    lane = lax.iota(jnp.int32, L)
    lane0 = lane == 0

    def _sst(ref, idx, val):
        plsc.store_scatter(ref, [jnp.full((L,), idx, jnp.int32)],
                           jnp.full((L,), val), mask=lane0)

    def _sstv(ref, idx, vec):
        plsc.store_scatter(ref, [jnp.full((L,), idx, jnp.int32)], vec,
                           mask=lane0)

    def _sldv(ref, idx):
        return plsc.load_gather(ref, [jnp.full((L,), idx, jnp.int32)])

    def _scal(x):
        return x[0] if getattr(x, "ndim", 0) else x

    blo = (NBLK * s) // NS
    bhi = (NBLK * (s + 1)) // NS
    fbhi = jnp.minimum(bhi, NFULL)
    b_lo = blo * 128

    negv16 = jnp.full((L,), NEG, jnp.float32)
    for v in range(80):
        scores_loc[pl.ds(L * v, L)] = negv16

    def _groups(buf, lbuf, ngroups, col0):
        def grp(g, _):
            best = buf[0, pl.ds(L * g, L)]
            bc = jnp.zeros((L,), jnp.int32)
            for j in range(1, C):
                v = buf[j, pl.ds(L * g, L)]
                sel = v > best
                best = jnp.where(sel, v, best)
                bc = jnp.where(sel, j, bc)
            scores_loc[pl.ds(col0 - b_lo + L * g, L)] = best
            lbuf[pl.ds(L * g, L)] = bc
            return 0

        lax.fori_loop(0, ngroups, grp, 0)

    def _cissue(blk_k, buf, sem):
        base_blk = jnp.minimum(blo + blk_k, fbhi - 1)
        pltpu.async_copy(
            cls_hbm.at[pl.ds(c * C, C), pl.ds(base_blk * 128, 128)],
            buf, sem)

    _cissue(0, ct_a, sem_a)
    _cissue(1, ct_b, sem_b)

    def chunk2(k2, _):
        for b, (buf, sem) in enumerate(((ct_a, sem_a), (ct_b, sem_b))):
            k = 2 * k2 + b
            pltpu.make_async_copy(
                cls_hbm.at[pl.ds(0, C), pl.ds(0, 128)], buf, sem).wait()
            base_blk = jnp.minimum(blo + k, fbhi - 1)
            col0 = base_blk * 128
            _groups(buf, lb_chunk, 8, col0)
            pltpu.sync_copy(lb_chunk, labels_sh.at[pl.ds(col0, 128)])

            @pl.when(k + 2 < NCHUNK)
            def _():
                _cissue(k + 2, buf, sem)

        return 0

    lax.fori_loop(0, NCHUNK // 2, chunk2, 0)

    @pl.when(s == NS - 1)
    def _tail():
        col0 = NFULL * 128
        pltpu.sync_copy(ctail_hbm, ct_tail)

        def trow(r, _):
            vecs = [ct_tail[c * 32 + r, pl.ds(L * j, L)] for j in range(5)]
            best = vecs[0]
            bc = lane
            for j in range(1, 5):
                sel = vecs[j] > best
                best = jnp.where(sel, vecs[j], best)
                bc = jnp.where(sel, lane + L * j, bc)
            rm = jnp.max(best)
            _sst(scores_loc, col0 - b_lo + r, rm)
            _sst(lb_chunk, r, jnp.min(jnp.where(best == rm, bc, 127)))
            return 0

        lax.fori_loop(0, 32, trow, 0)
        pltpu.sync_copy(lb_chunk.at[pl.ds(0, 32)],
                        labels_sh.at[pl.ds(col0, 32)])

    def cml(g, _):
        _sst(cm_loc, g, jnp.max(scores_loc[pl.ds(L * g, L)]))
        return 0

    lax.fori_loop(0, 80, cml, 0, unroll=4)

    def ltk(t, _):
        cvs = [cm_loc[pl.ds(L * v, L)] for v in range(5)]
        m = jnp.max(_vecmax5(cvs))
        g = jnp.int32(1 << 20)
        for v in range(5):
            eq = cvs[v] == m
            cnt = _scal(plsc.all_reduce_population_count(eq))
            ff = _scal(plsc.all_reduce_ffs(eq))
            g = jnp.minimum(g, jnp.where(cnt > 0, L * v + ff, 1 << 20))
        svec = scores_loc[pl.ds(L * g, L)]
        lfv = plsc.all_reduce_ffs(svec == m)
        winl = lane == lfv
        _sst(loc_sc, t, m)
        plsc.store_scatter(loc_idx, [jnp.full((L,), t, jnp.int32)],
                           b_lo + L * g + lane, mask=winl)
        svec2 = jnp.where(winl, NEG, svec)
        scores_loc[pl.ds(L * g, L)] = svec2
        _sst(cm_loc, g, jnp.max(svec2))
        return 0

    for v in range(19):
        loc_sc[pl.ds(L * v, L)] = negv16
    lax.fori_loop(0, K1, ltk, 0)
    pltpu.sync_copy(loc_sc, locsc_sh.at[pl.ds(TOPP * s, TOPP)])
    pltpu.sync_copy(loc_idx, locidx_sh.at[pl.ds(TOPP * s, TOPP)])

    plsc.subcore_barrier()

    def _do_merge():
        pos0 = jnp.zeros((L,), jnp.int32)
        heads0 = plsc.load_gather(msc, [lane * TOPP])
        hidx0 = plsc.load_gather(midx, [lane * TOPP])
        heads20 = plsc.load_gather(msc, [lane * TOPP + 1])
        hidx20 = plsc.load_gather(midx, [lane * TOPP + 1])

        def mg_body(t, carry):
            pos, heads, hidx, heads2, hidx2 = carry
            m = jnp.max(heads)
            win = lane == plsc.all_reduce_ffs(heads == m)
            tt = jnp.full((L,), t, jnp.int32)
            plsc.store_scatter(top_idx, [tt], hidx, mask=win)
            plsc.store_scatter(top_sc, [tt], heads, mask=win)
            pos = jnp.where(win, pos + 1, pos)
            heads = jnp.where(win, heads2, heads)
            hidx = jnp.where(win, hidx2, hidx)
            addr = lane * TOPP + pos + 1
            heads2 = jnp.where(win, plsc.load_gather(msc, [addr], mask=win),
                               heads2)
            hidx2 = jnp.where(win, plsc.load_gather(midx, [addr], mask=win),
                              hidx2)
            return (pos, heads, hidx, heads2, hidx2)

        pos, _, _, _, _ = lax.fori_loop(
            0, MAXDET, mg_body, (pos0, heads0, hidx0, heads20, hidx20))
        return pos

    @pl.when(s == 0)
    def _topk1():
        pltpu.sync_copy(locsc_sh, msc)
        pltpu.sync_copy(locidx_sh, midx)

        for v in range(18, 20):
            top_sc[pl.ds(L * v, L)] = jnp.full((L,), NEG, jnp.float32)
        zi = jnp.zeros((L,), jnp.int32)
        for v in range(19):
            top_idx[pl.ds(L * v, L)] = zi

        pos = _do_merge()
        drained = jnp.max(jnp.where(pos >= K1, 1, 0).astype(jnp.int32))
        fbuf[pl.ds(0, L)] = jnp.zeros((L,), jnp.int32) + drained
        pltpu.sync_copy(fbuf, flag_sh)

    plsc.subcore_barrier()
    pltpu.sync_copy(flag_sh, fbuf)
    bad = fbuf[pl.ds(0, L)][0]

    @pl.when(bad > 0)
    def _extend():
        lax.fori_loop(K1, MAXDET, ltk, 0)
        pltpu.sync_copy(loc_sc, locsc_sh.at[pl.ds(TOPP * s, TOPP)])
        pltpu.sync_copy(loc_idx, locidx_sh.at[pl.ds(TOPP * s, TOPP)])

    plsc.subcore_barrier()

    @pl.when(s == 0)
    def _topk2():
        @pl.when(bad > 0)
        def _remerge():
            pltpu.sync_copy(locsc_sh, msc)
            pltpu.sync_copy(locidx_sh, midx)
            _do_merge()

        rounds = [(2560 * q, 2560) for q in range(7)] + [(17920, 2048)]
        for col0, w in rounds:
            pltpu.async_copy(boxt_hbm.at[c, :, pl.ds(col0, w)],
                             box_buf.at[:, pl.ds(0, w)], sem_c).wait()
            for v in range(19):
                idxv = top_idx[pl.ds(L * v, L)]
                inq = (idxv >= col0) & (idxv < col0 + w)
                loc = jnp.clip(idxv - col0, 0, w - 1)
                for k in range(4):
                    vals = plsc.load_gather(
                        box_buf, [jnp.full((L,), k, jnp.int32), loc])
                    cur = fb_soa[pl.ds(TOPP * k + L * v, L)]
                    fb_soa[pl.ds(TOPP * k + L * v, L)] = (
                        jnp.where(inq, vals, cur))
        pltpu.sync_copy(btail_hbm, btail_buf)
        for v in range(19):
            idxv = top_idx[pl.ds(L * v, L)]
            inq = idxv >= NFULL * 128
            loc = jnp.clip(idxv - NFULL * 128, 0, 31)
            for k in range(4):
                vals = plsc.load_gather(
                    btail_buf, [jnp.full((L,), c * 4 + k, jnp.int32), loc])
                cur = fb_soa[pl.ds(TOPP * k + L * v, L)]
                fb_soa[pl.ds(TOPP * k + L * v, L)] = (
                    jnp.where(inq, vals, cur))

        acc = jnp.zeros((L,), jnp.int32)
        for v in range(19):
            vec = top_sc[pl.ds(L * v, L)]
            acc = acc + jnp.where(vec > SCORE_TH, 1, 0).astype(jnp.int32)
        nv_smem[0] = jnp.sum(acc)

        pltpu.sync_copy(fb_soa, fb_sh)

    plsc.subcore_barrier()

    @pl.when((s > 0) & (s <= 13))
    def _iou():
        pltpu.sync_copy(fb_sh, fb_soa)
        r0 = jnp.minimum((s - 1) * 24, TOPP - 24)

        def iou_row(rr, _):
            i = r0 + rr
            ax1 = _sldv(fb_soa, i)
            ay1 = _sldv(fb_soa, TOPP + i)
            ax2 = _sldv(fb_soa, 2 * TOPP + i)
            ay2 = _sldv(fb_soa, 3 * TOPP + i)
            area_a = (ax2 - ax1) * (ay2 - ay1)
            for v in range(19):
                bx1 = fb_soa[pl.ds(L * v, L)]
                by1 = fb_soa[pl.ds(TOPP + L * v, L)]
                bx2 = fb_soa[pl.ds(2 * TOPP + L * v, L)]
                by2 = fb_soa[pl.ds(3 * TOPP + L * v, L)]
                ltx = jnp.maximum(ax1, bx1)
                lty = jnp.maximum(ay1, by1)
                rbx = jnp.minimum(ax2, bx2)
                rby = jnp.minimum(ay2, by2)
                iw = jnp.maximum(rbx - ltx, 0.0)
                ih = jnp.maximum(rby - lty, 0.0)
                area_i = iw * ih
                area_b = (bx2 - bx1) * (by2 - by1)
                area_u = jnp.maximum(area_a + area_b - area_i, 1e-07)
                iou_loc[pl.ds(TOPP * rr + L * v, L)] = area_i / area_u
            return 0

        lax.fori_loop(0, 24, iou_row, 0)
        pltpu.sync_copy(iou_loc, iou_sh.at[pl.ds(r0 * TOPP, 24 * TOPP)])

    @pl.when(s == 0)
    def _labels():
        pltpu.sync_copy(labels_sh, labels_f)
        for v in range(19):
            idxv = top_idx[pl.ds(L * v, L)]
            labels_v[pl.ds(L * v, L)] = plsc.load_gather(labels_f, [idxv])

    plsc.subcore_barrier()

    @pl.when(s == 0)
    def _nms():
        nv = nv_smem[0]
        for v in range(19):
            col = lane + L * v
            alive[pl.ds(L * v, L)] = jnp.where(col < nv, 1, 0).astype(jnp.int32)
        negv = jnp.full((L,), -1.0, jnp.float32)
        negi = jnp.full((L,), -1, jnp.int32)
        for v in range(75):
            stage_b[pl.ds(L * v, L)] = negv
        for v in range(19):
            stage_s[pl.ds(L * v, L)] = negv
            stage_l[pl.ds(L * v, L)] = negi

        cnt = jnp.int32(0)
        for h in range(4):
            pltpu.sync_copy(iou_sh.at[pl.ds(NHALF * h * TOPP, NHALF * TOPP)],
                            iou_half)
            hi = jnp.minimum(nv, NHALF * (h + 1))

            def nms_i(i, cnt):
                def keep_fn(cc):
                    rbase = (i - NHALF * h) * TOPP

                    for v in range((NHALF * h) // L, 19):
                        iouv = iou_half[pl.ds(rbase + L * v, L)]
                        al = alive[pl.ds(L * v, L)]
                        col = lane + L * v
                        kill = (col > i) & (iouv >= NMS_TH)
                        alive[pl.ds(L * v, L)] = jnp.where(kill, 0, al)
                    _sstv(stage_b, 4 * cc + 0, _sldv(fb_soa, i))
                    _sstv(stage_b, 4 * cc + 1, _sldv(fb_soa, TOPP + i))
                    _sstv(stage_b, 4 * cc + 2, _sldv(fb_soa, 2 * TOPP + i))
                    _sstv(stage_b, 4 * cc + 3, _sldv(fb_soa, 3 * TOPP + i))
                    _sstv(stage_s, cc, _sldv(top_sc, i))
                    _sstv(stage_l, cc, _sldv(labels_v, i))
                    return cc + 1

                return lax.cond(_sldv(alive, i)[0] > 0, keep_fn,
                                lambda cc: cc, cnt)

            cnt = lax.fori_loop(NHALF * h, hi, nms_i, cnt)

        pltpu.sync_copy(stage_b, ob_hbm.at[pl.ds(c * MAXDET * 4, MAXDET * 4)])
        pltpu.sync_copy(stage_s, os_hbm.at[pl.ds(c * TOPP, TOPP)])
        pltpu.sync_copy(stage_l, ol_hbm.at[pl.ds(c * TOPP, TOPP)])


@jax.jit
def kernel(boxes, classification):
    boxt = boxes.transpose(0, 2, 1)
    clsT = classification.transpose(0, 2, 1).reshape(NB * C, N)
    ctail = classification[:, NFULL * 128:, :].reshape(2 * 32, C)
    btail = boxes[:, NFULL * 128:, :].transpose(0, 2, 1).reshape(8, 32)
    f32 = jnp.float32
    i32 = jnp.int32
    fd = pl.kernel(
        _fd_body,
        out_type=(
            jax.ShapeDtypeStruct((NB * MAXDET * 4,), f32),
            jax.ShapeDtypeStruct((NB * TOPP,), f32),
            jax.ShapeDtypeStruct((NB * TOPP,), i32),
        ),
        mesh=plsc.VectorSubcoreMesh(core_axis_name="c", subcore_axis_name="s"),
        compiler_params=pltpu.CompilerParams(needs_layout_passes=False),
        scratch_types=[
            pltpu.VMEM((C, 128), f32),
            pltpu.VMEM((C, 128), f32),
            pltpu.VMEM((2 * 32, C), f32),
            pltpu.VMEM((8, 32), f32),
            pltpu.VMEM((128,), i32),
            pltpu.VMEM((1280,), f32),
            pltpu.VMEM((80,), f32),
            pltpu.VMEM((TOPP,), f32),
            pltpu.VMEM((TOPP,), i32),
            pltpu.VMEM((NS * TOPP,), f32),
            pltpu.VMEM((NS * TOPP,), i32),
            pltpu.VMEM((N,), i32),
            pltpu.VMEM((TOPP,), i32),
            pltpu.VMEM((320,), f32),
            pltpu.VMEM((TOPP,), i32),
            pltpu.VMEM((4 * TOPP,), f32),
            pltpu.VMEM((4, 2560), f32),
            pltpu.VMEM((24 * TOPP,), f32),
            pltpu.VMEM((NHALF * TOPP,), f32),
            pltpu.VMEM((TOPP,), i32),
            pltpu.VMEM((MAXDET * 4,), f32),
            pltpu.VMEM((TOPP,), f32),
            pltpu.VMEM((TOPP,), i32),
            pltpu.VMEM((L,), i32),
            pltpu.SMEM((1,), i32),
            pltpu.VMEM_SHARED((NS * TOPP,), f32),
            pltpu.VMEM_SHARED((NS * TOPP,), i32),
            pltpu.VMEM_SHARED((N,), i32),
            pltpu.VMEM_SHARED((4 * TOPP,), f32),
            pltpu.VMEM_SHARED((TOPP * TOPP,), f32),
            pltpu.VMEM_SHARED((L,), i32),
            pltpu.SemaphoreType.DMA,
            pltpu.SemaphoreType.DMA,
            pltpu.SemaphoreType.DMA,
        ],
    )
    ob, os_, ol = fd(boxt, clsT, ctail, btail)
    return (ob.reshape(NB, MAXDET, 4),
            os_.reshape(NB, TOPP)[:, :MAXDET],
            ol.reshape(NB, TOPP)[:, :MAXDET])
